# baseline (speedup 1.0000x reference)
import functools

import jax
import jax.numpy as jnp
from jax import lax
from jax.experimental import pallas as pl
from jax.experimental.pallas import tpu as pltpu
from jax.experimental.pallas import tpu_sc as plsc

B = 16384
DM = 32
DF = 16
NC, NS = 2, 16
NW = NC * NS
BPW = B // NW
CHUNK = 128
NCH = BPW // CHUNK

_mesh = plsc.VectorSubcoreMesh(core_axis_name="c", subcore_axis_name="s")


@functools.partial(
    pl.kernel,
    out_type=(
        jax.ShapeDtypeStruct((B, DM), jnp.float32),
        jax.ShapeDtypeStruct((B, DM), jnp.float32),
        jax.ShapeDtypeStruct((B, DF), jnp.float32),
    ),
    mesh=_mesh,
    compiler_params=pltpu.CompilerParams(use_tc_tiling_on_sc=False),
    scratch_types=[
        pltpu.VMEM((BPW,), jnp.int32),
        pltpu.VMEM((BPW,), jnp.int32),
        pltpu.VMEM((BPW, DM), jnp.float32),
        pltpu.VMEM((BPW, DM), jnp.float32),
        pltpu.VMEM((BPW, DF), jnp.float32),
        pltpu.VMEM((BPW, DF), jnp.float32),
        pltpu.VMEM((BPW, DF), jnp.float32),
        pltpu.SemaphoreType.DMA,
        pltpu.SemaphoreType.DMA,
        pltpu.SemaphoreType.DMA,
        pltpu.SemaphoreType.DMA,
        pltpu.SemaphoreType.DMA,
    ],
)
def _sc_gather(gi_hbm, si_hbm, umlp_hbm, imlp_hbm, umf_hbm, imf_hbm,
               out_u, out_i, out_mf,
               gidx, sidx, bu, bi, bmu, bmi, bprod,
               sem_u, sem_i, sem_mu, sem_mi, sem_w):
    wid = lax.axis_index("s") * NC + lax.axis_index("c")
    base = wid * BPW
    pltpu.sync_copy(gi_hbm.at[pl.ds(base, BPW)], gidx)
    pltpu.sync_copy(si_hbm.at[pl.ds(base, BPW)], sidx)
    cps = []
    for j in range(NCH):
        s = pl.ds(j * CHUNK, CHUNK)
        cps.append(pltpu.async_copy(umlp_hbm.at[gidx.at[s]], bu.at[s], sem_u))
        cps.append(pltpu.async_copy(imlp_hbm.at[sidx.at[s]], bi.at[s], sem_i))
        cps.append(pltpu.async_copy(umf_hbm.at[gidx.at[s]], bmu.at[s], sem_mu))
        cps.append(pltpu.async_copy(imf_hbm.at[sidx.at[s]], bmi.at[s], sem_mi))
    for c in cps:
        c.wait()
    wu = pltpu.async_copy(bu, out_u.at[pl.ds(base, BPW)], sem_w)
    wi = pltpu.async_copy(bi, out_i.at[pl.ds(base, BPW)], sem_w)

    def body(r, carry):
        bprod[r] = bmu[r] * bmi[r]
        return carry

    lax.fori_loop(0, BPW, body, 0)
    wp = pltpu.async_copy(bprod, out_mf.at[pl.ds(base, BPW)], sem_w)
    wu.wait()
    wi.wait()
    wp.wait()


_GB = 2048


def _tc_body(xu_ref, xi_ref, mf_ref, w1u_ref, w1i_ref, b1_ref, w2_ref, b2_ref,
             wam_ref, waf_ref, ba_ref, o_ref):
    h = jnp.dot(xu_ref[...], w1u_ref[...], preferred_element_type=jnp.float32)
    h = h + jnp.dot(xi_ref[...], w1i_ref[...], preferred_element_type=jnp.float32)
    h = jnp.maximum(h + b1_ref[...], 0.0)
    h = jnp.dot(h, w2_ref[...], preferred_element_type=jnp.float32) + b2_ref[...]
    h = jnp.maximum(h, 0.0)
    lg = (jnp.sum(h * wam_ref[...], axis=1)
          + jnp.sum(mf_ref[...] * waf_ref[...], axis=1) + ba_ref[0])
    o_ref[...] = 1.0 / (1.0 + jnp.exp(-lg))


def _tc_mlp(xu, xi, mf, w1u, w1i, b1, w2, b2, wam, waf, ba):
    grid = (B // _GB,)
    full = lambda shape: pl.BlockSpec(shape, lambda i: (0,) * len(shape))
    return pl.pallas_call(
        _tc_body,
        grid=grid,
        in_specs=[
            pl.BlockSpec((_GB, DM), lambda i: (i, 0)),
            pl.BlockSpec((_GB, DM), lambda i: (i, 0)),
            pl.BlockSpec((_GB, DF), lambda i: (i, 0)),
            full((DM, DM)),
            full((DM, DM)),
            full((1, DM)),
            full((DM, DF)),
            full((1, DF)),
            full((1, DF)),
            full((1, DF)),
            pl.BlockSpec(memory_space=pltpu.SMEM),
        ],
        out_specs=pl.BlockSpec((_GB,), lambda i: (i,)),
        out_shape=jax.ShapeDtypeStruct((B,), jnp.float32),
    )(xu, xi, mf, w1u, w1i, b1, w2, b2, wam, waf, ba)


def kernel(gene_indices, spot_indices, emb_user_mlp, emb_item_mlp,
           emb_user_mf, emb_item_mf, W1, b1, W2, b2, Wa, ba):
    gi = gene_indices.astype(jnp.int32)
    si = spot_indices.astype(jnp.int32)
    xu, xi, mf = _sc_gather(gi, si, emb_user_mlp, emb_item_mlp,
                            emb_user_mf, emb_item_mf)
    return _tc_mlp(
        xu, xi, mf,
        W1[:DM], W1[DM:], b1.reshape(1, DM),
        W2, b2.reshape(1, DF),
        Wa[:DF, 0].reshape(1, DF), Wa[DF:, 0].reshape(1, DF),
        ba.reshape(1),
    )


# speedup vs baseline: 1.0414x; 1.0414x over previous
import functools

import jax
import jax.numpy as jnp
from jax import lax
from jax.experimental import pallas as pl
from jax.experimental.pallas import tpu as pltpu
from jax.experimental.pallas import tpu_sc as plsc

B = 16384
DM = 32
DF = 16
NC, NS = 2, 16
NW = NC * NS
BPW = B // NW
CHUNK = 128
NCH = BPW // CHUNK

NITEM = 1000001
NTF = NITEM // 128
TAIL = NITEM - NTF * 128
NTT = NTF
NPM = NTT * 32
NPF = NTT * 16

_mesh = plsc.VectorSubcoreMesh(core_axis_name="c", subcore_axis_name="s")
_nolayout = pltpu.CompilerParams(needs_layout_passes=False)


@functools.partial(
    pl.kernel,
    out_type=(jax.ShapeDtypeStruct((B, DM), jnp.float32),
              jax.ShapeDtypeStruct((B, DF), jnp.float32)),
    mesh=_mesh,
    compiler_params=pltpu.CompilerParams(use_tc_tiling_on_sc=False),
    scratch_types=[
        pltpu.VMEM((BPW,), jnp.int32),
        pltpu.VMEM((BPW, DM), jnp.float32),
        pltpu.VMEM((BPW, DF), jnp.float32),
        pltpu.SemaphoreType.DMA,
        pltpu.SemaphoreType.DMA,
    ],
)
def _sc_user(gi_hbm, umlp_hbm, umf_hbm, out_u, out_mfu,
             gidx, bu, bmu, sem_g, sem_w):
    wid = lax.axis_index("s") * NC + lax.axis_index("c")
    base = wid * BPW
    pltpu.sync_copy(gi_hbm.at[pl.ds(base, BPW)], gidx)
    cps = []
    for j in range(NCH):
        s = pl.ds(j * CHUNK, CHUNK)
        cps.append(pltpu.async_copy(umlp_hbm.at[gidx.at[s]], bu.at[s], sem_g))
        cps.append(pltpu.async_copy(umf_hbm.at[gidx.at[s]], bmu.at[s], sem_g))
    for c in cps:
        c.wait()
    w1 = pltpu.async_copy(bu, out_u.at[pl.ds(base, BPW)], sem_w)
    w2 = pltpu.async_copy(bmu, out_mfu.at[pl.ds(base, BPW)], sem_w)
    w1.wait()
    w2.wait()


@functools.partial(
    pl.kernel,
    out_type=(jax.ShapeDtypeStruct((NPM * 128,), jnp.float32),
              jax.ShapeDtypeStruct((NPF * 128,), jnp.float32)),
    mesh=_mesh,
    compiler_params=_nolayout,
    scratch_types=[
        pltpu.VMEM((DM, 128), jnp.float32),
        pltpu.VMEM((DM, 128), jnp.float32),
        pltpu.VMEM((DF, 128), jnp.float32),
        pltpu.VMEM((DF, 128), jnp.float32),
        pltpu.VMEM((32 * 128,), jnp.float32),
        pltpu.VMEM((32 * 128,), jnp.float32),
        pltpu.VMEM((16 * 128,), jnp.float32),
        pltpu.VMEM((16 * 128,), jnp.float32),
        pltpu.SemaphoreType.DMA,
        pltpu.SemaphoreType.DMA,
        pltpu.SemaphoreType.DMA,
        pltpu.SemaphoreType.DMA,
    ],
)
def _sc_pack(imlpT_hbm, imfT_hbm, out_pm, out_pf,
             tm0, tm1, tf0, tf1, pkm0, pkm1, pkf0, pkf1,
             sr0, sr1, sw0, sw1):
    tms, tfs, pkms, pkfs = (tm0, tm1), (tf0, tf1), (pkm0, pkm1), (pkf0, pkf1)
    srs, sws = (sr0, sr1), (sw0, sw1)
    wid = lax.axis_index("s") * NC + lax.axis_index("c")
    lanes = lax.iota(jnp.int32, 16)
    perm4 = (lanes // 4) * 128 + (lanes % 4) * 32
    perm8 = (lanes // 8) * 128 + (lanes % 8) * 16
    nt = (NTT - wid + NW - 1) // NW

    def fetch(t, b):
        a = pl.multiple_of(t * 128, 128)
        pltpu.async_copy(imlpT_hbm.at[:, pl.ds(a, 128)], tms[b], srs[b])
        pltpu.async_copy(imfT_hbm.at[:, pl.ds(a, 128)], tfs[b], srs[b])

    def wait_fetch(t, b):
        pltpu.make_async_copy(imlpT_hbm.at[:, pl.ds(0, 128)], tms[b], srs[b]).wait()
        pltpu.make_async_copy(imfT_hbm.at[:, pl.ds(0, 128)], tfs[b], srs[b]).wait()

    fetch(wid, 0)

    @pl.when(wid + NW < NTT)
    def _():
        fetch(wid + NW, 1)

    npair = (nt + 1) // 2

    def body(i, carry):
        for b in range(2):
            t = wid + (2 * i + b) * NW

            @pl.when(t < NTT)
            def _(t=t, b=b):
                wait_fetch(t, b)

                @pl.when(2 * i + b >= 2)
                def _(b=b):
                    pltpu.make_async_copy(pkms[b], out_pm.at[pl.ds(0, 32 * 128)], sws[b]).wait()
                    pltpu.make_async_copy(pkfs[b], out_pf.at[pl.ds(0, 16 * 128)], sws[b]).wait()

                @pl.when(t + 2 * NW < NTT)
                def _(t=t, b=b):
                    fetch(t + 2 * NW, b)

                for c in range(DM):
                    for k in range(8):
                        v = tms[b][c, pl.ds(16 * k, 16)]
                        plsc.store_scatter(pkms[b],
                                           [perm4 + (512 * k + c)], v)
                for c in range(DF):
                    for k in range(8):
                        v = tfs[b][c, pl.ds(16 * k, 16)]
                        plsc.store_scatter(pkfs[b],
                                           [perm8 + (256 * k + c)], v)
                pltpu.async_copy(pkms[b], out_pm.at[pl.ds(t * 4096, 4096)], sws[b])
                pltpu.async_copy(pkfs[b], out_pf.at[pl.ds(t * 2048, 2048)], sws[b])
        return carry

    lax.fori_loop(0, npair, body, 0)
    for b in range(2):
        @pl.when(nt > b)
        def _(b=b):
            pltpu.make_async_copy(pkms[b], out_pm.at[pl.ds(0, 32 * 128)], sws[b]).wait()
            pltpu.make_async_copy(pkfs[b], out_pf.at[pl.ds(0, 16 * 128)], sws[b]).wait()


NWAVE = 8
WV = BPW // NWAVE

@functools.partial(
    pl.kernel,
    out_type=(jax.ShapeDtypeStruct((B, DM), jnp.float32),
              jax.ShapeDtypeStruct((B, DF), jnp.float32)),
    mesh=_mesh,
    compiler_params=_nolayout,
    scratch_types=[
        pltpu.VMEM((BPW,), jnp.int32),
        pltpu.VMEM((BPW,), jnp.int32),
        pltpu.VMEM((BPW,), jnp.int32),
        pltpu.VMEM((BPW,), jnp.int32),
        pltpu.VMEM((BPW,), jnp.int32),
        pltpu.VMEM((2 * WV, 128), jnp.float32),
        pltpu.VMEM((2 * WV, 128), jnp.float32),
        pltpu.VMEM((WV, DM), jnp.float32),
        pltpu.VMEM((WV, DM), jnp.float32),
        pltpu.VMEM((WV, DF), jnp.float32),
        pltpu.VMEM((WV, DF), jnp.float32),
        pltpu.VMEM((TAIL * DM,), jnp.float32),
        pltpu.VMEM((TAIL * DF,), jnp.float32),
        pltpu.SemaphoreType.DMA((2,)),
        pltpu.SemaphoreType.DMA,
        pltpu.SemaphoreType.DMA,
    ],
)
def _sc_item(si_hbm, pm_hbm, pf_hbm, tlm_hbm, tlf_hbm, out_i, out_mfi,
             sidx, pim, pif, ph4, ph8, bm, bf, rm0, rm1, rf0, rf1, tlm, tlf,
             sem_r, sw0, sw1):
    rms, rfs, sws = (rm0, rm1), (rf0, rf1), (sw0, sw1)
    wid = lax.axis_index("s") * NC + lax.axis_index("c")
    base = wid * BPW
    pltpu.sync_copy(si_hbm.at[pl.ds(base, BPW)], sidx)
    pltpu.sync_copy(tlm_hbm, tlm)
    pltpu.sync_copy(tlf_hbm, tlf)
    for k in range(BPW // 16):
        s = pl.ds(16 * k, 16)
        v = sidx[s]
        vc = jnp.minimum(v, NTF * 128 - 1)
        pim[s] = vc >> 2
        pif[s] = vc >> 3
        ph4[s] = (vc & 3) * 32
        ph8[s] = (vc & 7) * 16
    lanes = lax.iota(jnp.int32, 16)

    def fetch(w):
        h = w % 2
        s = pl.ds(w * WV, WV)
        d = pl.ds(h * WV, WV)
        pltpu.async_copy(pm_hbm.at[pim.at[s]], bm.at[d], sem_r.at[h])
        pltpu.async_copy(pf_hbm.at[pif.at[s]], bf.at[d], sem_r.at[h])

    fetch(0)
    fetch(1)
    for w in range(NWAVE):
        h = w % 2
        pltpu.make_async_copy(pm_hbm.at[pim.at[pl.ds(0, WV)]],
                              bm.at[pl.ds(h * WV, WV)], sem_r.at[h]).wait()
        pltpu.make_async_copy(pf_hbm.at[pif.at[pl.ds(0, WV)]],
                              bf.at[pl.ds(h * WV, WV)], sem_r.at[h]).wait()
        if w >= 2:
            pltpu.make_async_copy(rms[h], out_i.at[pl.ds(0, WV)],
                                  sws[h]).wait()
            pltpu.make_async_copy(rfs[h], out_mfi.at[pl.ds(0, WV)],
                                  sws[h]).wait()

        def ebody(g, carry, w=w, h=h):
            s = w * WV + g
            sv = jnp.full((16,), s, jnp.int32)
            b4 = plsc.load_gather(ph4, [sv])
            b8 = plsc.load_gather(ph8, [sv])
            sloc = h * WV + g
            v0 = plsc.load_gather(bm.at[sloc], [lanes + b4])
            v1 = plsc.load_gather(bm.at[sloc], [lanes + 16 + b4])
            v2 = plsc.load_gather(bf.at[sloc], [lanes + b8])
            rv = plsc.load_gather(sidx, [sv])
            tmask = rv >= NTF * 128
            toff = jnp.maximum(rv - NTF * 128, 0)
            t0 = plsc.load_gather(tlm, [toff * DM + lanes])
            t1 = plsc.load_gather(tlm, [toff * DM + 16 + lanes])
            t2 = plsc.load_gather(tlf, [toff * DF + lanes])
            rms[h][g, pl.ds(0, 16)] = jnp.where(tmask, t0, v0)
            rms[h][g, pl.ds(16, 16)] = jnp.where(tmask, t1, v1)
            rfs[h][g, pl.ds(0, 16)] = jnp.where(tmask, t2, v2)
            return carry

        lax.fori_loop(0, WV, ebody, 0)
        pltpu.async_copy(rms[h], out_i.at[pl.ds(base + w * WV, WV)], sws[h])
        pltpu.async_copy(rfs[h], out_mfi.at[pl.ds(base + w * WV, WV)], sws[h])
        if w + 2 < NWAVE:
            fetch(w + 2)
    for h in range(2):
        pltpu.make_async_copy(rms[h], out_i.at[pl.ds(0, WV)], sws[h]).wait()
        pltpu.make_async_copy(rfs[h], out_mfi.at[pl.ds(0, WV)], sws[h]).wait()


_GB = 2048


def _tc_body(xu_ref, xi_ref, mfu_ref, mfi_ref, w1u_ref, w1i_ref, b1_ref,
             w2_ref, b2_ref, wam_ref, waf_ref, ba_ref, o_ref):
    h = jnp.dot(xu_ref[...], w1u_ref[...], preferred_element_type=jnp.float32)
    h = h + jnp.dot(xi_ref[...], w1i_ref[...],
                    preferred_element_type=jnp.float32)
    h = jnp.maximum(h + b1_ref[...], 0.0)
    h = jnp.dot(h, w2_ref[...], preferred_element_type=jnp.float32) + b2_ref[...]
    h = jnp.maximum(h, 0.0)
    mf = mfu_ref[...] * mfi_ref[...]
    lg = (jnp.sum(h * wam_ref[...], axis=1)
          + jnp.sum(mf * waf_ref[...], axis=1) + ba_ref[0])
    o_ref[...] = 1.0 / (1.0 + jnp.exp(-lg))


def _tc_mlp(xu, xi, mfu, mfi, w1u, w1i, b1, w2, b2, wam, waf, ba):
    grid = (B // _GB,)
    full = lambda shape: pl.BlockSpec(shape, lambda i: (0,) * len(shape))
    return pl.pallas_call(
        _tc_body,
        grid=grid,
        in_specs=[
            pl.BlockSpec((_GB, DM), lambda i: (i, 0)),
            pl.BlockSpec((_GB, DM), lambda i: (i, 0)),
            pl.BlockSpec((_GB, DF), lambda i: (i, 0)),
            pl.BlockSpec((_GB, DF), lambda i: (i, 0)),
            full((DM, DM)),
            full((DM, DM)),
            full((1, DM)),
            full((DM, DF)),
            full((1, DF)),
            full((1, DF)),
            full((1, DF)),
            pl.BlockSpec(memory_space=pltpu.SMEM),
        ],
        out_specs=pl.BlockSpec((_GB,), lambda i: (i,)),
        out_shape=jax.ShapeDtypeStruct((B,), jnp.float32),
    )(xu, xi, mfu, mfi, w1u, w1i, b1, w2, b2, wam, waf, ba)


def kernel(gene_indices, spot_indices, emb_user_mlp, emb_item_mlp,
           emb_user_mf, emb_item_mf, W1, b1, W2, b2, Wa, ba):
    gi = gene_indices.astype(jnp.int32)
    si = spot_indices.astype(jnp.int32)
    xu, mfu = _sc_user(gi, emb_user_mlp, emb_user_mf)
    pm1, pf1 = _sc_pack(emb_item_mlp.T, emb_item_mf.T)
    pm = pm1.reshape(NPM, 128)
    pf = pf1.reshape(NPF, 128)
    tlm = emb_item_mlp[NTF * 128:].reshape(-1)
    tlf = emb_item_mf[NTF * 128:].reshape(-1)
    xi, mfi = _sc_item(si, pm, pf, tlm, tlf)
    return _tc_mlp(
        xu, xi, mfu, mfi,
        W1[:DM], W1[DM:], b1.reshape(1, DM),
        W2, b2.reshape(1, DF),
        Wa[:DF, 0].reshape(1, DF), Wa[DF:, 0].reshape(1, DF),
        ba.reshape(1),
    )


# speedup vs baseline: 1.1724x; 1.1257x over previous
import functools

import jax
import jax.numpy as jnp
from jax import lax
from jax.experimental import pallas as pl
from jax.experimental.pallas import tpu as pltpu
from jax.experimental.pallas import tpu_sc as plsc

B = 16384
DM = 32
DF = 16
NC, NS = 2, 16
NW = NC * NS
BPW = B // NW
CHUNK = 128
NCH = BPW // CHUNK

NITEM = 1000001
NTF = NITEM // 128
TAIL = NITEM - NTF * 128
NTT = NTF
NPM = NTT * 32
NPF = NTT * 16

_mesh = plsc.VectorSubcoreMesh(core_axis_name="c", subcore_axis_name="s")
_nolayout = pltpu.CompilerParams(needs_layout_passes=False)


@functools.partial(
    pl.kernel,
    out_type=(jax.ShapeDtypeStruct((B, DM), jnp.float32),
              jax.ShapeDtypeStruct((B, DF), jnp.float32)),
    mesh=_mesh,
    compiler_params=pltpu.CompilerParams(use_tc_tiling_on_sc=False),
    scratch_types=[
        pltpu.VMEM((BPW,), jnp.int32),
        pltpu.VMEM((BPW, DM), jnp.float32),
        pltpu.VMEM((BPW, DF), jnp.float32),
        pltpu.SemaphoreType.DMA,
        pltpu.SemaphoreType.DMA,
    ],
)
def _sc_user(gi_hbm, umlp_hbm, umf_hbm, out_u, out_mfu,
             gidx, bu, bmu, sem_g, sem_w):
    wid = lax.axis_index("s") * NC + lax.axis_index("c")
    base = wid * BPW
    pltpu.sync_copy(gi_hbm.at[pl.ds(base, BPW)], gidx)
    cps = []
    for j in range(NCH):
        s = pl.ds(j * CHUNK, CHUNK)
        cps.append(pltpu.async_copy(umlp_hbm.at[gidx.at[s]], bu.at[s], sem_g))
        cps.append(pltpu.async_copy(umf_hbm.at[gidx.at[s]], bmu.at[s], sem_g))
    for c in cps:
        c.wait()
    w1 = pltpu.async_copy(bu, out_u.at[pl.ds(base, BPW)], sem_w)
    w2 = pltpu.async_copy(bmu, out_mfu.at[pl.ds(base, BPW)], sem_w)
    w1.wait()
    w2.wait()


@functools.partial(
    pl.kernel,
    out_type=(jax.ShapeDtypeStruct((NPM * 128,), jnp.float32),
              jax.ShapeDtypeStruct((NPF * 128,), jnp.float32)),
    mesh=_mesh,
    compiler_params=_nolayout,
    scratch_types=[
        pltpu.VMEM((DM, 128), jnp.float32),
        pltpu.VMEM((DM, 128), jnp.float32),
        pltpu.VMEM((DF, 128), jnp.float32),
        pltpu.VMEM((DF, 128), jnp.float32),
        pltpu.VMEM((32 * 128,), jnp.float32),
        pltpu.VMEM((32 * 128,), jnp.float32),
        pltpu.VMEM((16 * 128,), jnp.float32),
        pltpu.VMEM((16 * 128,), jnp.float32),
        pltpu.SemaphoreType.DMA,
        pltpu.SemaphoreType.DMA,
        pltpu.SemaphoreType.DMA,
        pltpu.SemaphoreType.DMA,
    ],
)
def _sc_pack(imlpT_hbm, imfT_hbm, out_pm, out_pf,
             tm0, tm1, tf0, tf1, pkm0, pkm1, pkf0, pkf1,
             sr0, sr1, sw0, sw1):
    tms, tfs, pkms, pkfs = (tm0, tm1), (tf0, tf1), (pkm0, pkm1), (pkf0, pkf1)
    srs, sws = (sr0, sr1), (sw0, sw1)
    wid = lax.axis_index("s") * NC + lax.axis_index("c")
    lanes = lax.iota(jnp.int32, 16)
    nt = (NTT - wid + NW - 1) // NW

    def fetch(t, b):
        a = pl.multiple_of(t * 128, 128)
        pltpu.async_copy(imlpT_hbm.at[:, pl.ds(a, 128)], tms[b], srs[b])
        pltpu.async_copy(imfT_hbm.at[:, pl.ds(a, 128)], tfs[b], srs[b])

    def wait_fetch(t, b):
        pltpu.make_async_copy(imlpT_hbm.at[:, pl.ds(0, 128)], tms[b], srs[b]).wait()
        pltpu.make_async_copy(imfT_hbm.at[:, pl.ds(0, 128)], tfs[b], srs[b]).wait()

    fetch(wid, 0)

    @pl.when(wid + NW < NTT)
    def _():
        fetch(wid + NW, 1)

    npair = (nt + 1) // 2

    def body(i, carry):
        for b in range(2):
            t = wid + (2 * i + b) * NW

            @pl.when(t < NTT)
            def _(t=t, b=b):
                wait_fetch(t, b)

                @pl.when(2 * i + b >= 2)
                def _(b=b):
                    pltpu.make_async_copy(pkms[b], out_pm.at[pl.ds(0, 32 * 128)], sws[b]).wait()
                    pltpu.make_async_copy(pkfs[b], out_pf.at[pl.ds(0, 16 * 128)], sws[b]).wait()

                @pl.when(t + 2 * NW < NTT)
                def _(t=t, b=b):
                    fetch(t + 2 * NW, b)

                for c in range(DM):
                    for k in range(8):
                        v = tms[b][c, pl.ds(16 * k, 16)]
                        idx = (512 * k + (lanes // 4) * 128
                               + ((4 * c + lanes % 4 + 16 * k
                                   + 4 * (lanes // 4)) & 127))
                        plsc.store_scatter(pkms[b], [idx], v)
                for c in range(DF):
                    for k in range(8):
                        v = tfs[b][c, pl.ds(16 * k, 16)]
                        idx = (256 * k + (lanes // 8) * 128
                               + ((8 * c + lanes % 8 + 16 * k
                                   + 8 * (lanes // 8)) & 127))
                        plsc.store_scatter(pkfs[b], [idx], v)
                pltpu.async_copy(pkms[b], out_pm.at[pl.ds(t * 4096, 4096)], sws[b])
                pltpu.async_copy(pkfs[b], out_pf.at[pl.ds(t * 2048, 2048)], sws[b])
        return carry

    lax.fori_loop(0, npair, body, 0)
    for b in range(2):
        @pl.when(nt > b)
        def _(b=b):
            pltpu.make_async_copy(pkms[b], out_pm.at[pl.ds(0, 32 * 128)], sws[b]).wait()
            pltpu.make_async_copy(pkfs[b], out_pf.at[pl.ds(0, 16 * 128)], sws[b]).wait()


NWAVE = 8
WV = BPW // NWAVE

@functools.partial(
    pl.kernel,
    out_type=(jax.ShapeDtypeStruct((B, DM), jnp.float32),
              jax.ShapeDtypeStruct((B, DF), jnp.float32)),
    mesh=_mesh,
    compiler_params=_nolayout,
    scratch_types=[
        pltpu.VMEM((BPW,), jnp.int32),
        pltpu.VMEM((BPW,), jnp.int32),
        pltpu.VMEM((BPW,), jnp.int32),
        pltpu.VMEM((BPW,), jnp.int32),
        pltpu.VMEM((BPW,), jnp.int32),
        pltpu.VMEM((2 * WV, 128), jnp.float32),
        pltpu.VMEM((2 * WV, 128), jnp.float32),
        pltpu.VMEM((WV, DM), jnp.float32),
        pltpu.VMEM((WV, DM), jnp.float32),
        pltpu.VMEM((WV, DF), jnp.float32),
        pltpu.VMEM((WV, DF), jnp.float32),
        pltpu.VMEM((TAIL * DM,), jnp.float32),
        pltpu.VMEM((TAIL * DF,), jnp.float32),
        pltpu.SemaphoreType.DMA((2,)),
        pltpu.SemaphoreType.DMA,
        pltpu.SemaphoreType.DMA,
    ],
)
def _sc_item(si_hbm, pm_hbm, pf_hbm, tlm_hbm, tlf_hbm, out_i, out_mfi,
             sidx, pim, pif, ph4, ph8, bm, bf, rm0, rm1, rf0, rf1, tlm, tlf,
             sem_r, sw0, sw1):
    rms, rfs, sws = (rm0, rm1), (rf0, rf1), (sw0, sw1)
    wid = lax.axis_index("s") * NC + lax.axis_index("c")
    base = wid * BPW
    pltpu.sync_copy(si_hbm.at[pl.ds(base, BPW)], sidx)
    pltpu.sync_copy(tlm_hbm, tlm)
    pltpu.sync_copy(tlf_hbm, tlf)
    for k in range(BPW // 16):
        s = pl.ds(16 * k, 16)
        v = sidx[s]
        vc = jnp.minimum(v, NTF * 128 - 1)
        pim[s] = vc >> 2
        pif[s] = vc >> 3
        ph4[s] = vc & 127
        ph8[s] = vc & 127
    lanes = lax.iota(jnp.int32, 16)

    def fetch(w):
        h = w % 2
        s = pl.ds(w * WV, WV)
        d = pl.ds(h * WV, WV)
        pltpu.async_copy(pm_hbm.at[pim.at[s]], bm.at[d], sem_r.at[h])
        pltpu.async_copy(pf_hbm.at[pif.at[s]], bf.at[d], sem_r.at[h])

    fetch(0)
    fetch(1)
    for w in range(NWAVE):
        h = w % 2
        pltpu.make_async_copy(pm_hbm.at[pim.at[pl.ds(0, WV)]],
                              bm.at[pl.ds(h * WV, WV)], sem_r.at[h]).wait()
        pltpu.make_async_copy(pf_hbm.at[pif.at[pl.ds(0, WV)]],
                              bf.at[pl.ds(h * WV, WV)], sem_r.at[h]).wait()
        if w >= 2:
            pltpu.make_async_copy(rms[h], out_i.at[pl.ds(0, WV)],
                                  sws[h]).wait()
            pltpu.make_async_copy(rfs[h], out_mfi.at[pl.ds(0, WV)],
                                  sws[h]).wait()

        def ebody(g, carry, w=w, h=h):
            s = w * WV + g
            sv = jnp.full((16,), s, jnp.int32)
            b4 = plsc.load_gather(ph4, [sv])
            b8 = plsc.load_gather(ph8, [sv])
            sloc = h * WV + g
            v0 = plsc.load_gather(bm.at[sloc], [(lanes * 4 + b4) & 127])
            v1 = plsc.load_gather(bm.at[sloc],
                                  [(lanes * 4 + 64 + b4) & 127])
            v2 = plsc.load_gather(bf.at[sloc], [(lanes * 8 + b8) & 127])
            rv = plsc.load_gather(sidx, [sv])
            tmask = rv >= NTF * 128
            toff = jnp.maximum(rv - NTF * 128, 0)
            t0 = plsc.load_gather(tlm, [toff * DM + lanes])
            t1 = plsc.load_gather(tlm, [toff * DM + 16 + lanes])
            t2 = plsc.load_gather(tlf, [toff * DF + lanes])
            rms[h][g, pl.ds(0, 16)] = jnp.where(tmask, t0, v0)
            rms[h][g, pl.ds(16, 16)] = jnp.where(tmask, t1, v1)
            rfs[h][g, pl.ds(0, 16)] = jnp.where(tmask, t2, v2)
            return carry

        lax.fori_loop(0, WV, ebody, 0)
        pltpu.async_copy(rms[h], out_i.at[pl.ds(base + w * WV, WV)], sws[h])
        pltpu.async_copy(rfs[h], out_mfi.at[pl.ds(base + w * WV, WV)], sws[h])
        if w + 2 < NWAVE:
            fetch(w + 2)
    for h in range(2):
        pltpu.make_async_copy(rms[h], out_i.at[pl.ds(0, WV)], sws[h]).wait()
        pltpu.make_async_copy(rfs[h], out_mfi.at[pl.ds(0, WV)], sws[h]).wait()


_GB = 2048


def _tc_body(xu_ref, xi_ref, mfu_ref, mfi_ref, w1u_ref, w1i_ref, b1_ref,
             w2_ref, b2_ref, wam_ref, waf_ref, ba_ref, o_ref):
    h = jnp.dot(xu_ref[...], w1u_ref[...], preferred_element_type=jnp.float32)
    h = h + jnp.dot(xi_ref[...], w1i_ref[...],
                    preferred_element_type=jnp.float32)
    h = jnp.maximum(h + b1_ref[...], 0.0)
    h = jnp.dot(h, w2_ref[...], preferred_element_type=jnp.float32) + b2_ref[...]
    h = jnp.maximum(h, 0.0)
    mf = mfu_ref[...] * mfi_ref[...]
    lg = (jnp.sum(h * wam_ref[...], axis=1)
          + jnp.sum(mf * waf_ref[...], axis=1) + ba_ref[0])
    o_ref[...] = 1.0 / (1.0 + jnp.exp(-lg))


def _tc_mlp(xu, xi, mfu, mfi, w1u, w1i, b1, w2, b2, wam, waf, ba):
    grid = (B // _GB,)
    full = lambda shape: pl.BlockSpec(shape, lambda i: (0,) * len(shape))
    return pl.pallas_call(
        _tc_body,
        grid=grid,
        in_specs=[
            pl.BlockSpec((_GB, DM), lambda i: (i, 0)),
            pl.BlockSpec((_GB, DM), lambda i: (i, 0)),
            pl.BlockSpec((_GB, DF), lambda i: (i, 0)),
            pl.BlockSpec((_GB, DF), lambda i: (i, 0)),
            full((DM, DM)),
            full((DM, DM)),
            full((1, DM)),
            full((DM, DF)),
            full((1, DF)),
            full((1, DF)),
            full((1, DF)),
            pl.BlockSpec(memory_space=pltpu.SMEM),
        ],
        out_specs=pl.BlockSpec((_GB,), lambda i: (i,)),
        out_shape=jax.ShapeDtypeStruct((B,), jnp.float32),
    )(xu, xi, mfu, mfi, w1u, w1i, b1, w2, b2, wam, waf, ba)


def kernel(gene_indices, spot_indices, emb_user_mlp, emb_item_mlp,
           emb_user_mf, emb_item_mf, W1, b1, W2, b2, Wa, ba):
    gi = gene_indices.astype(jnp.int32)
    si = spot_indices.astype(jnp.int32)
    xu, mfu = _sc_user(gi, emb_user_mlp, emb_user_mf)
    pm1, pf1 = _sc_pack(emb_item_mlp.T, emb_item_mf.T)
    pm = pm1.reshape(NPM, 128)
    pf = pf1.reshape(NPF, 128)
    tlm = emb_item_mlp[NTF * 128:].reshape(-1)
    tlf = emb_item_mf[NTF * 128:].reshape(-1)
    xi, mfi = _sc_item(si, pm, pf, tlm, tlf)
    return _tc_mlp(
        xu, xi, mfu, mfi,
        W1[:DM], W1[DM:], b1.reshape(1, DM),
        W2, b2.reshape(1, DF),
        Wa[:DF, 0].reshape(1, DF), Wa[DF:, 0].reshape(1, DF),
        ba.reshape(1),
    )


# speedup vs baseline: 1.1914x; 1.0163x over previous
import functools

import jax
import jax.numpy as jnp
from jax import lax
from jax.experimental import pallas as pl
from jax.experimental.pallas import tpu as pltpu
from jax.experimental.pallas import tpu_sc as plsc

B = 16384
DM = 32
DF = 16
NC, NS = 2, 16
NW = NC * NS
BPW = B // NW
CHUNK = 128
NCH = BPW // CHUNK

NITEM = 1000001
NTF = NITEM // 128
TAIL = NITEM - NTF * 128
NTT = NTF
NPM = NTT * 32
NPF = NTT * 16

_mesh = plsc.VectorSubcoreMesh(core_axis_name="c", subcore_axis_name="s")
_nolayout = pltpu.CompilerParams(needs_layout_passes=False)


@functools.partial(
    pl.kernel,
    out_type=(jax.ShapeDtypeStruct((B, DM), jnp.float32),
              jax.ShapeDtypeStruct((B, DF), jnp.float32)),
    mesh=_mesh,
    compiler_params=pltpu.CompilerParams(use_tc_tiling_on_sc=False),
    scratch_types=[
        pltpu.VMEM((BPW,), jnp.int32),
        pltpu.VMEM((BPW, DM), jnp.float32),
        pltpu.VMEM((BPW, DF), jnp.float32),
        pltpu.SemaphoreType.DMA,
        pltpu.SemaphoreType.DMA,
    ],
)
def _sc_user(gi_hbm, umlp_hbm, umf_hbm, out_u, out_mfu,
             gidx, bu, bmu, sem_g, sem_w):
    wid = lax.axis_index("s") * NC + lax.axis_index("c")
    base = wid * BPW
    pltpu.sync_copy(gi_hbm.at[pl.ds(base, BPW)], gidx)
    cps = []
    for j in range(NCH):
        s = pl.ds(j * CHUNK, CHUNK)
        cps.append(pltpu.async_copy(umlp_hbm.at[gidx.at[s]], bu.at[s], sem_g))
        cps.append(pltpu.async_copy(umf_hbm.at[gidx.at[s]], bmu.at[s], sem_g))
    for c in cps:
        c.wait()
    w1 = pltpu.async_copy(bu, out_u.at[pl.ds(base, BPW)], sem_w)
    w2 = pltpu.async_copy(bmu, out_mfu.at[pl.ds(base, BPW)], sem_w)
    w1.wait()
    w2.wait()


@functools.partial(
    pl.kernel,
    out_type=(jax.ShapeDtypeStruct((NPM * 128,), jnp.float32),
              jax.ShapeDtypeStruct((NPF * 128,), jnp.float32)),
    mesh=_mesh,
    compiler_params=_nolayout,
    scratch_types=[
        pltpu.VMEM((DM, 128), jnp.float32),
        pltpu.VMEM((DM, 128), jnp.float32),
        pltpu.VMEM((DF, 128), jnp.float32),
        pltpu.VMEM((DF, 128), jnp.float32),
        pltpu.VMEM((32 * 128,), jnp.float32),
        pltpu.VMEM((32 * 128,), jnp.float32),
        pltpu.VMEM((16 * 128,), jnp.float32),
        pltpu.VMEM((16 * 128,), jnp.float32),
        pltpu.SemaphoreType.DMA,
        pltpu.SemaphoreType.DMA,
        pltpu.SemaphoreType.DMA,
        pltpu.SemaphoreType.DMA,
    ],
)
def _sc_pack(imlpT_hbm, imfT_hbm, out_pm, out_pf,
             tm0, tm1, tf0, tf1, pkm0, pkm1, pkf0, pkf1,
             sr0, sr1, sw0, sw1):
    tms, tfs, pkms, pkfs = (tm0, tm1), (tf0, tf1), (pkm0, pkm1), (pkf0, pkf1)
    srs, sws = (sr0, sr1), (sw0, sw1)
    wid = lax.axis_index("s") * NC + lax.axis_index("c")
    lanes = lax.iota(jnp.int32, 16)
    nt = (NTT - wid + NW - 1) // NW

    def fetch(t, b):
        a = pl.multiple_of(t * 128, 128)
        pltpu.async_copy(imlpT_hbm.at[:, pl.ds(a, 128)], tms[b], srs[b])
        pltpu.async_copy(imfT_hbm.at[:, pl.ds(a, 128)], tfs[b], srs[b])

    def wait_fetch(t, b):
        pltpu.make_async_copy(imlpT_hbm.at[:, pl.ds(0, 128)], tms[b], srs[b]).wait()
        pltpu.make_async_copy(imfT_hbm.at[:, pl.ds(0, 128)], tfs[b], srs[b]).wait()

    fetch(wid, 0)

    @pl.when(wid + NW < NTT)
    def _():
        fetch(wid + NW, 1)

    npair = (nt + 1) // 2

    def body(i, carry):
        for b in range(2):
            t = wid + (2 * i + b) * NW

            @pl.when(t < NTT)
            def _(t=t, b=b):
                wait_fetch(t, b)

                @pl.when(2 * i + b >= 2)
                def _(b=b):
                    pltpu.make_async_copy(pkms[b], out_pm.at[pl.ds(0, 32 * 128)], sws[b]).wait()
                    pltpu.make_async_copy(pkfs[b], out_pf.at[pl.ds(0, 16 * 128)], sws[b]).wait()

                @pl.when(t + 2 * NW < NTT)
                def _(t=t, b=b):
                    fetch(t + 2 * NW, b)

                for k in range(8):
                    base4 = 512 * k + (lanes // 4) * 128
                    mix4 = (lanes + 16 * k) & 127
                    for c in range(DM):
                        v = tms[b][c, pl.ds(16 * k, 16)]
                        idx = base4 + ((mix4 + 4 * c) & 127)
                        plsc.store_scatter(pkms[b], [idx], v)
                for k in range(8):
                    base8 = 256 * k + (lanes // 8) * 128
                    mix8 = (lanes % 8 + 16 * k + 8 * (lanes // 8)) & 127
                    for c in range(DF):
                        v = tfs[b][c, pl.ds(16 * k, 16)]
                        idx = base8 + ((mix8 + 8 * c) & 127)
                        plsc.store_scatter(pkfs[b], [idx], v)
                pltpu.async_copy(pkms[b], out_pm.at[pl.ds(t * 4096, 4096)], sws[b])
                pltpu.async_copy(pkfs[b], out_pf.at[pl.ds(t * 2048, 2048)], sws[b])
        return carry

    lax.fori_loop(0, npair, body, 0)
    for b in range(2):
        @pl.when(nt > b)
        def _(b=b):
            pltpu.make_async_copy(pkms[b], out_pm.at[pl.ds(0, 32 * 128)], sws[b]).wait()
            pltpu.make_async_copy(pkfs[b], out_pf.at[pl.ds(0, 16 * 128)], sws[b]).wait()


NWAVE = 8
WV = BPW // NWAVE

@functools.partial(
    pl.kernel,
    out_type=(jax.ShapeDtypeStruct((B, DM), jnp.float32),
              jax.ShapeDtypeStruct((B, DF), jnp.float32)),
    mesh=_mesh,
    compiler_params=_nolayout,
    scratch_types=[
        pltpu.VMEM((BPW,), jnp.int32),
        pltpu.VMEM((BPW,), jnp.int32),
        pltpu.VMEM((BPW,), jnp.int32),
        pltpu.VMEM((BPW,), jnp.int32),
        pltpu.VMEM((BPW,), jnp.int32),
        pltpu.VMEM((2 * WV, 128), jnp.float32),
        pltpu.VMEM((2 * WV, 128), jnp.float32),
        pltpu.VMEM((WV, DM), jnp.float32),
        pltpu.VMEM((WV, DM), jnp.float32),
        pltpu.VMEM((WV, DF), jnp.float32),
        pltpu.VMEM((WV, DF), jnp.float32),
        pltpu.VMEM((TAIL * DM,), jnp.float32),
        pltpu.VMEM((TAIL * DF,), jnp.float32),
        pltpu.SemaphoreType.DMA((2,)),
        pltpu.SemaphoreType.DMA,
        pltpu.SemaphoreType.DMA,
    ],
)
def _sc_item(si_hbm, pm_hbm, pf_hbm, tlm_hbm, tlf_hbm, out_i, out_mfi,
             sidx, pim, pif, ph4, ph8, bm, bf, rm0, rm1, rf0, rf1, tlm, tlf,
             sem_r, sw0, sw1):
    rms, rfs, sws = (rm0, rm1), (rf0, rf1), (sw0, sw1)
    wid = lax.axis_index("s") * NC + lax.axis_index("c")
    base = wid * BPW
    pltpu.sync_copy(si_hbm.at[pl.ds(base, BPW)], sidx)
    pltpu.sync_copy(tlm_hbm, tlm)
    pltpu.sync_copy(tlf_hbm, tlf)
    for k in range(BPW // 16):
        s = pl.ds(16 * k, 16)
        v = sidx[s]
        vc = jnp.minimum(v, NTF * 128 - 1)
        pim[s] = vc >> 2
        pif[s] = vc >> 3
        ph4[s] = vc & 127
        ph8[s] = vc & 127
    lanes = lax.iota(jnp.int32, 16)

    def fetch(w):
        h = w % 2
        s = pl.ds(w * WV, WV)
        d = pl.ds(h * WV, WV)
        pltpu.async_copy(pm_hbm.at[pim.at[s]], bm.at[d], sem_r.at[h])
        pltpu.async_copy(pf_hbm.at[pif.at[s]], bf.at[d], sem_r.at[h])

    fetch(0)
    fetch(1)
    for w in range(NWAVE):
        h = w % 2
        pltpu.make_async_copy(pm_hbm.at[pim.at[pl.ds(0, WV)]],
                              bm.at[pl.ds(h * WV, WV)], sem_r.at[h]).wait()
        pltpu.make_async_copy(pf_hbm.at[pif.at[pl.ds(0, WV)]],
                              bf.at[pl.ds(h * WV, WV)], sem_r.at[h]).wait()
        if w >= 2:
            pltpu.make_async_copy(rms[h], out_i.at[pl.ds(0, WV)],
                                  sws[h]).wait()
            pltpu.make_async_copy(rfs[h], out_mfi.at[pl.ds(0, WV)],
                                  sws[h]).wait()

        def ebody(g, carry, w=w, h=h):
            s = w * WV + g
            sv = jnp.full((16,), s, jnp.int32)
            b4 = plsc.load_gather(ph4, [sv])
            b8 = plsc.load_gather(ph8, [sv])
            sloc = h * WV + g
            v0 = plsc.load_gather(bm.at[sloc], [(lanes * 4 + b4) & 127])
            v1 = plsc.load_gather(bm.at[sloc],
                                  [(lanes * 4 + 64 + b4) & 127])
            v2 = plsc.load_gather(bf.at[sloc], [(lanes * 8 + b8) & 127])
            rv = plsc.load_gather(sidx, [sv])
            tmask = rv >= NTF * 128
            toff = jnp.maximum(rv - NTF * 128, 0)
            t0 = plsc.load_gather(tlm, [toff * DM + lanes])
            t1 = plsc.load_gather(tlm, [toff * DM + 16 + lanes])
            t2 = plsc.load_gather(tlf, [toff * DF + lanes])
            rms[h][g, pl.ds(0, 16)] = jnp.where(tmask, t0, v0)
            rms[h][g, pl.ds(16, 16)] = jnp.where(tmask, t1, v1)
            rfs[h][g, pl.ds(0, 16)] = jnp.where(tmask, t2, v2)
            return carry

        lax.fori_loop(0, WV, ebody, 0)
        pltpu.async_copy(rms[h], out_i.at[pl.ds(base + w * WV, WV)], sws[h])
        pltpu.async_copy(rfs[h], out_mfi.at[pl.ds(base + w * WV, WV)], sws[h])
        if w + 2 < NWAVE:
            fetch(w + 2)
    for h in range(2):
        pltpu.make_async_copy(rms[h], out_i.at[pl.ds(0, WV)], sws[h]).wait()
        pltpu.make_async_copy(rfs[h], out_mfi.at[pl.ds(0, WV)], sws[h]).wait()


_GB = 2048


def _tc_body(xu_ref, xi_ref, mfu_ref, mfi_ref, w1u_ref, w1i_ref, b1_ref,
             w2_ref, b2_ref, wam_ref, waf_ref, ba_ref, o_ref):
    h = jnp.dot(xu_ref[...], w1u_ref[...], preferred_element_type=jnp.float32)
    h = h + jnp.dot(xi_ref[...], w1i_ref[...],
                    preferred_element_type=jnp.float32)
    h = jnp.maximum(h + b1_ref[...], 0.0)
    h = jnp.dot(h, w2_ref[...], preferred_element_type=jnp.float32) + b2_ref[...]
    h = jnp.maximum(h, 0.0)
    mf = mfu_ref[...] * mfi_ref[...]
    lg = (jnp.sum(h * wam_ref[...], axis=1)
          + jnp.sum(mf * waf_ref[...], axis=1) + ba_ref[0])
    o_ref[...] = 1.0 / (1.0 + jnp.exp(-lg))


def _tc_mlp(xu, xi, mfu, mfi, w1u, w1i, b1, w2, b2, wam, waf, ba):
    grid = (B // _GB,)
    full = lambda shape: pl.BlockSpec(shape, lambda i: (0,) * len(shape))
    return pl.pallas_call(
        _tc_body,
        grid=grid,
        in_specs=[
            pl.BlockSpec((_GB, DM), lambda i: (i, 0)),
            pl.BlockSpec((_GB, DM), lambda i: (i, 0)),
            pl.BlockSpec((_GB, DF), lambda i: (i, 0)),
            pl.BlockSpec((_GB, DF), lambda i: (i, 0)),
            full((DM, DM)),
            full((DM, DM)),
            full((1, DM)),
            full((DM, DF)),
            full((1, DF)),
            full((1, DF)),
            full((1, DF)),
            pl.BlockSpec(memory_space=pltpu.SMEM),
        ],
        out_specs=pl.BlockSpec((_GB,), lambda i: (i,)),
        out_shape=jax.ShapeDtypeStruct((B,), jnp.float32),
    )(xu, xi, mfu, mfi, w1u, w1i, b1, w2, b2, wam, waf, ba)


def kernel(gene_indices, spot_indices, emb_user_mlp, emb_item_mlp,
           emb_user_mf, emb_item_mf, W1, b1, W2, b2, Wa, ba):
    gi = gene_indices.astype(jnp.int32)
    si = spot_indices.astype(jnp.int32)
    xu, mfu = _sc_user(gi, emb_user_mlp, emb_user_mf)
    pm1, pf1 = _sc_pack(emb_item_mlp.T, emb_item_mf.T)
    pm = pm1.reshape(NPM, 128)
    pf = pf1.reshape(NPF, 128)
    tlm = emb_item_mlp[NTF * 128:].reshape(-1)
    tlf = emb_item_mf[NTF * 128:].reshape(-1)
    xi, mfi = _sc_item(si, pm, pf, tlm, tlf)
    return _tc_mlp(
        xu, xi, mfu, mfi,
        W1[:DM], W1[DM:], b1.reshape(1, DM),
        W2, b2.reshape(1, DF),
        Wa[:DF, 0].reshape(1, DF), Wa[DF:, 0].reshape(1, DF),
        ba.reshape(1),
    )


# speedup vs baseline: 3.4332x; 2.8815x over previous
import functools

import jax
import jax.numpy as jnp
from jax import lax
from jax.experimental import pallas as pl
from jax.experimental.pallas import tpu as pltpu
from jax.experimental.pallas import tpu_sc as plsc

B = 16384
DM = 32
DF = 16
NC, NS = 2, 16
NW = NC * NS
BPW = B // NW
CHUNK = 128
NCH = BPW // CHUNK

NITEM = 1000001
NTF = NITEM // 128
TAIL = NITEM - NTF * 128
NTT = NTF
NPM = NTT * 32
NPF = NTT * 16

_mesh = plsc.VectorSubcoreMesh(core_axis_name="c", subcore_axis_name="s")
_nolayout = pltpu.CompilerParams(needs_layout_passes=False)


@functools.partial(
    pl.kernel,
    out_type=(jax.ShapeDtypeStruct((B, DM), jnp.float32),
              jax.ShapeDtypeStruct((B, DF), jnp.float32)),
    mesh=_mesh,
    compiler_params=pltpu.CompilerParams(use_tc_tiling_on_sc=False),
    scratch_types=[
        pltpu.VMEM((BPW,), jnp.int32),
        pltpu.VMEM((BPW, DM), jnp.float32),
        pltpu.VMEM((BPW, DF), jnp.float32),
        pltpu.SemaphoreType.DMA,
        pltpu.SemaphoreType.DMA,
    ],
)
def _sc_user(gi_hbm, umlp_hbm, umf_hbm, out_u, out_mfu,
             gidx, bu, bmu, sem_g, sem_w):
    wid = lax.axis_index("s") * NC + lax.axis_index("c")
    base = wid * BPW
    pltpu.sync_copy(gi_hbm.at[pl.ds(base, BPW)], gidx)
    cps = []
    for j in range(NCH):
        s = pl.ds(j * CHUNK, CHUNK)
        cps.append(pltpu.async_copy(umlp_hbm.at[gidx.at[s]], bu.at[s], sem_g))
        cps.append(pltpu.async_copy(umf_hbm.at[gidx.at[s]], bmu.at[s], sem_g))
    for c in cps:
        c.wait()
    w1 = pltpu.async_copy(bu, out_u.at[pl.ds(base, BPW)], sem_w)
    w2 = pltpu.async_copy(bmu, out_mfu.at[pl.ds(base, BPW)], sem_w)
    w1.wait()
    w2.wait()


RING = 8


@functools.partial(
    pl.kernel,
    out_type=(jax.ShapeDtypeStruct((B * DM,), jnp.float32),
              jax.ShapeDtypeStruct((B * DF,), jnp.float32)),
    mesh=_mesh,
    compiler_params=_nolayout,
    scratch_types=(
        [pltpu.VMEM((BPW,), jnp.int32)]
        + [pltpu.VMEM((DM, 128), jnp.float32) for _ in range(RING)]
        + [pltpu.VMEM((DF, 128), jnp.float32) for _ in range(RING)]
        + [pltpu.VMEM((DM,), jnp.float32) for _ in range(RING)]
        + [pltpu.VMEM((DF,), jnp.float32) for _ in range(RING)]
        + [
            pltpu.VMEM((TAIL * DM,), jnp.float32),
            pltpu.VMEM((TAIL * DF,), jnp.float32),
        ]
        + [pltpu.SemaphoreType.DMA for _ in range(2 * RING)]
    ),
)
def _sc_item(si_hbm, imlpT_hbm, imfT_hbm, tlm_hbm, tlf_hbm, out_i, out_mfi,
             sidx, *rest):
    bms = rest[0:RING]
    bfs = rest[RING:2 * RING]
    rmb = rest[2 * RING:3 * RING]
    rfb = rest[3 * RING:4 * RING]
    tlm, tlf = rest[4 * RING:4 * RING + 2]
    srs = rest[4 * RING + 2:5 * RING + 2]
    sws = rest[5 * RING + 2:6 * RING + 2]
    wid = lax.axis_index("s") * NC + lax.axis_index("c")
    base = wid * BPW
    pltpu.sync_copy(si_hbm.at[pl.ds(base, BPW)], sidx)
    pltpu.sync_copy(tlm_hbm, tlm)
    pltpu.sync_copy(tlf_hbm, tlf)
    lanes = lax.iota(jnp.int32, 16)

    def fetch(s, slot):
        sv = jnp.full((16,), s, jnp.int32)
        rsp = plsc.load_gather(sidx, [sv])
        rc = jnp.minimum(rsp, NTF * 128 - 1)
        r = jnp.max(rc)
        a = pl.multiple_of((r // 128) * 128, 128)
        pltpu.async_copy(imlpT_hbm.at[:, pl.ds(a, 128)], bms[slot], srs[slot])
        pltpu.async_copy(imfT_hbm.at[:, pl.ds(a, 128)], bfs[slot], srs[slot])

    for slot in range(RING):
        fetch(slot, slot)

    zero16 = jnp.zeros((16,), jnp.float32)

    def body(o, carry):
        for slot in range(RING):
            s = o * RING + slot
            pltpu.make_async_copy(imlpT_hbm.at[:, pl.ds(0, 128)], bms[slot],
                                  srs[slot]).wait()
            pltpu.make_async_copy(imfT_hbm.at[:, pl.ds(0, 128)], bfs[slot],
                                  srs[slot]).wait()

            @pl.when(o >= 1)
            def _(slot=slot):
                pltpu.make_async_copy(rmb[slot], out_i.at[pl.ds(0, DM)],
                                      sws[slot]).wait()
                pltpu.make_async_copy(rfb[slot], out_mfi.at[pl.ds(0, DF)],
                                      sws[slot]).wait()

            sv = jnp.full((16,), s, jnp.int32)
            rsp = plsc.load_gather(sidx, [sv])
            rcp = jnp.minimum(rsp, NTF * 128 - 1)
            colv = rcp & 127
            tmask = rsp >= NTF * 128
            toff = jnp.maximum(rsp - NTF * 128, 0)
            v0 = zero16
            v1 = zero16
            v2 = zero16
            for c in range(DM):
                meq = lanes == (c % 16)
                g = plsc.load_gather(bms[slot].at[c], [colv], mask=meq)
                if c < 16:
                    v0 = jnp.where(meq, g, v0)
                else:
                    v1 = jnp.where(meq, g, v1)
            for c in range(DF):
                meq = lanes == c
                g = plsc.load_gather(bfs[slot].at[c], [colv], mask=meq)
                v2 = jnp.where(meq, g, v2)
            t0 = plsc.load_gather(tlm, [toff * DM + lanes])
            t1 = plsc.load_gather(tlm, [toff * DM + 16 + lanes])
            t2 = plsc.load_gather(tlf, [toff * DF + lanes])
            rmb[slot][pl.ds(0, 16)] = jnp.where(tmask, t0, v0)
            rmb[slot][pl.ds(16, 16)] = jnp.where(tmask, t1, v1)
            rfb[slot][pl.ds(0, 16)] = jnp.where(tmask, t2, v2)
            pltpu.async_copy(rmb[slot],
                             out_i.at[pl.ds((base + s) * DM, DM)], sws[slot])
            pltpu.async_copy(rfb[slot],
                             out_mfi.at[pl.ds((base + s) * DF, DF)],
                             sws[slot])

            @pl.when(s + RING < BPW)
            def _(s=s, slot=slot):
                fetch(s + RING, slot)

        return carry

    lax.fori_loop(0, BPW // RING, body, 0)
    for slot in range(RING):
        pltpu.make_async_copy(rmb[slot], out_i.at[pl.ds(0, DM)],
                              sws[slot]).wait()
        pltpu.make_async_copy(rfb[slot], out_mfi.at[pl.ds(0, DF)],
                              sws[slot]).wait()


_GB = 2048


def _tc_body(xu_ref, xi_ref, mfu_ref, mfi_ref, w1u_ref, w1i_ref, b1_ref,
             w2_ref, b2_ref, wam_ref, waf_ref, ba_ref, o_ref):
    h = jnp.dot(xu_ref[...], w1u_ref[...], preferred_element_type=jnp.float32)
    h = h + jnp.dot(xi_ref[...], w1i_ref[...],
                    preferred_element_type=jnp.float32)
    h = jnp.maximum(h + b1_ref[...], 0.0)
    h = jnp.dot(h, w2_ref[...], preferred_element_type=jnp.float32) + b2_ref[...]
    h = jnp.maximum(h, 0.0)
    mf = mfu_ref[...] * mfi_ref[...]
    lg = (jnp.sum(h * wam_ref[...], axis=1)
          + jnp.sum(mf * waf_ref[...], axis=1) + ba_ref[0])
    o_ref[...] = 1.0 / (1.0 + jnp.exp(-lg))


def _tc_mlp(xu, xi, mfu, mfi, w1u, w1i, b1, w2, b2, wam, waf, ba):
    grid = (B // _GB,)
    full = lambda shape: pl.BlockSpec(shape, lambda i: (0,) * len(shape))
    return pl.pallas_call(
        _tc_body,
        grid=grid,
        in_specs=[
            pl.BlockSpec((_GB, DM), lambda i: (i, 0)),
            pl.BlockSpec((_GB, DM), lambda i: (i, 0)),
            pl.BlockSpec((_GB, DF), lambda i: (i, 0)),
            pl.BlockSpec((_GB, DF), lambda i: (i, 0)),
            full((DM, DM)),
            full((DM, DM)),
            full((1, DM)),
            full((DM, DF)),
            full((1, DF)),
            full((1, DF)),
            full((1, DF)),
            pl.BlockSpec(memory_space=pltpu.SMEM),
        ],
        out_specs=pl.BlockSpec((_GB,), lambda i: (i,)),
        out_shape=jax.ShapeDtypeStruct((B,), jnp.float32),
    )(xu, xi, mfu, mfi, w1u, w1i, b1, w2, b2, wam, waf, ba)


def kernel(gene_indices, spot_indices, emb_user_mlp, emb_item_mlp,
           emb_user_mf, emb_item_mf, W1, b1, W2, b2, Wa, ba):
    gi = gene_indices.astype(jnp.int32)
    si = spot_indices.astype(jnp.int32)
    xu, mfu = _sc_user(gi, emb_user_mlp, emb_user_mf)
    tlm = emb_item_mlp[NTF * 128:].reshape(-1)
    tlf = emb_item_mf[NTF * 128:].reshape(-1)
    xi1, mfi1 = _sc_item(si, emb_item_mlp.T, emb_item_mf.T, tlm, tlf)
    xi = xi1.reshape(B, DM)
    mfi = mfi1.reshape(B, DF)
    return _tc_mlp(
        xu, xi, mfu, mfi,
        W1[:DM], W1[DM:], b1.reshape(1, DM),
        W2, b2.reshape(1, DF),
        Wa[:DF, 0].reshape(1, DF), Wa[DF:, 0].reshape(1, DF),
        ba.reshape(1),
    )


# speedup vs baseline: 3.4423x; 1.0027x over previous
import functools

import jax
import jax.numpy as jnp
from jax import lax
from jax.experimental import pallas as pl
from jax.experimental.pallas import tpu as pltpu
from jax.experimental.pallas import tpu_sc as plsc

B = 16384
DM = 32
DF = 16
NC, NS = 2, 16
NW = NC * NS
BPW = B // NW
CHUNK = 128
NCH = BPW // CHUNK

NITEM = 1000001
NTF = NITEM // 128
TAIL = NITEM - NTF * 128
NTT = NTF
NPM = NTT * 32
NPF = NTT * 16

_mesh = plsc.VectorSubcoreMesh(core_axis_name="c", subcore_axis_name="s")
_nolayout = pltpu.CompilerParams(needs_layout_passes=False)


@functools.partial(
    pl.kernel,
    out_type=(jax.ShapeDtypeStruct((B, DM), jnp.float32),
              jax.ShapeDtypeStruct((B, DF), jnp.float32)),
    mesh=_mesh,
    compiler_params=pltpu.CompilerParams(use_tc_tiling_on_sc=False),
    scratch_types=[
        pltpu.VMEM((BPW,), jnp.int32),
        pltpu.VMEM((BPW, DM), jnp.float32),
        pltpu.VMEM((BPW, DF), jnp.float32),
        pltpu.SemaphoreType.DMA,
        pltpu.SemaphoreType.DMA,
    ],
)
def _sc_user(gi_hbm, umlp_hbm, umf_hbm, out_u, out_mfu,
             gidx, bu, bmu, sem_g, sem_w):
    wid = lax.axis_index("s") * NC + lax.axis_index("c")
    base = wid * BPW
    pltpu.sync_copy(gi_hbm.at[pl.ds(base, BPW)], gidx)
    cps = []
    for j in range(NCH):
        s = pl.ds(j * CHUNK, CHUNK)
        cps.append(pltpu.async_copy(umlp_hbm.at[gidx.at[s]], bu.at[s], sem_g))
        cps.append(pltpu.async_copy(umf_hbm.at[gidx.at[s]], bmu.at[s], sem_g))
    for c in cps:
        c.wait()
    w1 = pltpu.async_copy(bu, out_u.at[pl.ds(base, BPW)], sem_w)
    w2 = pltpu.async_copy(bmu, out_mfu.at[pl.ds(base, BPW)], sem_w)
    w1.wait()
    w2.wait()


RING = 8


@functools.partial(
    pl.kernel,
    out_type=(jax.ShapeDtypeStruct((B * DM,), jnp.float32),
              jax.ShapeDtypeStruct((B * DF,), jnp.float32)),
    mesh=_mesh,
    compiler_params=_nolayout,
    scratch_types=(
        [pltpu.VMEM((BPW,), jnp.int32)]
        + [pltpu.VMEM((DM, 128), jnp.float32) for _ in range(RING)]
        + [pltpu.VMEM((DF, 128), jnp.float32) for _ in range(RING)]
        + [pltpu.VMEM((DM,), jnp.float32) for _ in range(RING)]
        + [pltpu.VMEM((DF,), jnp.float32) for _ in range(RING)]
        + [
            pltpu.VMEM((TAIL * DM,), jnp.float32),
            pltpu.VMEM((TAIL * DF,), jnp.float32),
        ]
        + [pltpu.SemaphoreType.DMA for _ in range(2 * RING)]
    ),
)
def _sc_item(si_hbm, imlpT_hbm, imfT_hbm, tlm_hbm, tlf_hbm, out_i, out_mfi,
             sidx, *rest):
    bms = rest[0:RING]
    bfs = rest[RING:2 * RING]
    rmb = rest[2 * RING:3 * RING]
    rfb = rest[3 * RING:4 * RING]
    tlm, tlf = rest[4 * RING:4 * RING + 2]
    srs = rest[4 * RING + 2:5 * RING + 2]
    sws = rest[5 * RING + 2:6 * RING + 2]
    wid = lax.axis_index("s") * NC + lax.axis_index("c")
    base = wid * BPW
    pltpu.sync_copy(si_hbm.at[pl.ds(base, BPW)], sidx)
    pltpu.sync_copy(tlm_hbm, tlm)
    pltpu.sync_copy(tlf_hbm, tlf)
    lanes = lax.iota(jnp.int32, 16)

    def fetch(s, slot):
        sv = jnp.full((16,), s, jnp.int32)
        rsp = plsc.load_gather(sidx, [sv])
        rc = jnp.minimum(rsp, NTF * 128 - 1)
        r = jnp.max(rc)
        a = pl.multiple_of((r // 128) * 128, 128)
        pltpu.async_copy(imlpT_hbm.at[:, pl.ds(a, 128)], bms[slot], srs[slot])
        pltpu.async_copy(imfT_hbm.at[:, pl.ds(a, 128)], bfs[slot], srs[slot])

    for slot in range(RING):
        fetch(slot, slot)

    zero16 = jnp.zeros((16,), jnp.float32)

    def body(o, carry):
        for slot in range(RING):
            s = o * RING + slot
            pltpu.make_async_copy(imlpT_hbm.at[:, pl.ds(0, 128)], bms[slot],
                                  srs[slot]).wait()
            pltpu.make_async_copy(imfT_hbm.at[:, pl.ds(0, 128)], bfs[slot],
                                  srs[slot]).wait()

            @pl.when(o >= 1)
            def _(slot=slot):
                pltpu.make_async_copy(rmb[slot], out_i.at[pl.ds(0, DM)],
                                      sws[slot]).wait()
                pltpu.make_async_copy(rfb[slot], out_mfi.at[pl.ds(0, DF)],
                                      sws[slot]).wait()

            sv = jnp.full((16,), s, jnp.int32)
            rsp = plsc.load_gather(sidx, [sv])
            rcp = jnp.minimum(rsp, NTF * 128 - 1)
            colv = rcp & 127
            tmask = rsp >= NTF * 128
            toff = jnp.maximum(rsp - NTF * 128, 0)
            v0 = zero16
            v1 = zero16
            v2 = zero16
            for c in range(DM):
                meq = lanes == (c % 16)
                g = plsc.load_gather(bms[slot].at[c], [colv], mask=meq)
                if c < 16:
                    v0 = jnp.where(meq, g, v0)
                else:
                    v1 = jnp.where(meq, g, v1)
            for c in range(DF):
                meq = lanes == c
                g = plsc.load_gather(bfs[slot].at[c], [colv], mask=meq)
                v2 = jnp.where(meq, g, v2)
            t0 = plsc.load_gather(tlm, [toff * DM + lanes])
            t1 = plsc.load_gather(tlm, [toff * DM + 16 + lanes])
            t2 = plsc.load_gather(tlf, [toff * DF + lanes])
            rmb[slot][pl.ds(0, 16)] = jnp.where(tmask, t0, v0)
            rmb[slot][pl.ds(16, 16)] = jnp.where(tmask, t1, v1)
            rfb[slot][pl.ds(0, 16)] = jnp.where(tmask, t2, v2)
            pltpu.async_copy(rmb[slot],
                             out_i.at[pl.ds((base + s) * DM, DM)], sws[slot])
            pltpu.async_copy(rfb[slot],
                             out_mfi.at[pl.ds((base + s) * DF, DF)],
                             sws[slot])

            @pl.when(s + RING < BPW)
            def _(s=s, slot=slot):
                fetch(s + RING, slot)

        return carry

    lax.fori_loop(0, BPW // RING, body, 0)
    for slot in range(RING):
        pltpu.make_async_copy(rmb[slot], out_i.at[pl.ds(0, DM)],
                              sws[slot]).wait()
        pltpu.make_async_copy(rfb[slot], out_mfi.at[pl.ds(0, DF)],
                              sws[slot]).wait()


_GB = 2048


def _tc_body(xu_ref, xi_ref, mfu_ref, mfi_ref, w1u_ref, w1i_ref, b1_ref,
             w2_ref, b2_ref, wam_ref, waf_ref, ba_ref, o_ref):
    h = jnp.dot(xu_ref[...], w1u_ref[...], preferred_element_type=jnp.float32)
    h = h + jnp.dot(xi_ref[...], w1i_ref[...],
                    preferred_element_type=jnp.float32)
    h = jnp.maximum(h + b1_ref[...], 0.0)
    h = jnp.dot(h, w2_ref[...], preferred_element_type=jnp.float32) + b2_ref[...]
    h = jnp.maximum(h, 0.0)
    mf = mfu_ref[...] * mfi_ref[...]
    lg = (jnp.dot(h, wam_ref[...], preferred_element_type=jnp.float32)
          + jnp.dot(mf, waf_ref[...], preferred_element_type=jnp.float32))
    lg = lg[:, 0] + ba_ref[0]
    o_ref[...] = 1.0 / (1.0 + jnp.exp(-lg))


def _tc_mlp(xu, xi, mfu, mfi, w1u, w1i, b1, w2, b2, wam, waf, ba):
    grid = (B // _GB,)
    full = lambda shape: pl.BlockSpec(shape, lambda i: (0,) * len(shape))
    return pl.pallas_call(
        _tc_body,
        grid=grid,
        in_specs=[
            pl.BlockSpec((_GB, DM), lambda i: (i, 0)),
            pl.BlockSpec((_GB, DM), lambda i: (i, 0)),
            pl.BlockSpec((_GB, DF), lambda i: (i, 0)),
            pl.BlockSpec((_GB, DF), lambda i: (i, 0)),
            full((DM, DM)),
            full((DM, DM)),
            full((1, DM)),
            full((DM, DF)),
            full((1, DF)),
            full((DF, 1)),
            full((DF, 1)),
            pl.BlockSpec(memory_space=pltpu.SMEM),
        ],
        out_specs=pl.BlockSpec((_GB,), lambda i: (i,)),
        out_shape=jax.ShapeDtypeStruct((B,), jnp.float32),
    )(xu, xi, mfu, mfi, w1u, w1i, b1, w2, b2, wam, waf, ba)


def kernel(gene_indices, spot_indices, emb_user_mlp, emb_item_mlp,
           emb_user_mf, emb_item_mf, W1, b1, W2, b2, Wa, ba):
    gi = gene_indices.astype(jnp.int32)
    si = spot_indices.astype(jnp.int32)
    xu, mfu = _sc_user(gi, emb_user_mlp, emb_user_mf)
    tlm = emb_item_mlp[NTF * 128:].reshape(-1)
    tlf = emb_item_mf[NTF * 128:].reshape(-1)
    xi1, mfi1 = _sc_item(si, emb_item_mlp.T, emb_item_mf.T, tlm, tlf)
    xi = xi1.reshape(B, DM)
    mfi = mfi1.reshape(B, DF)
    return _tc_mlp(
        xu, xi, mfu, mfi,
        W1[:DM], W1[DM:], b1.reshape(1, DM),
        W2, b2.reshape(1, DF),
        Wa[:DF], Wa[DF:],
        ba.reshape(1),
    )
